# Initial kernel scaffold; baseline (speedup 1.0000x reference)
import jax
import jax.numpy as jnp
from jax.experimental import pallas as pl


def kernel(x_list, edge_index_list, Wq, bq, Wk, bk, Wv, bv, ln_g, ln_b, W1, b1, W2, b2):
    raise NotImplementedError("write your pallas kernel here")


# speedup vs baseline: 2.9305x; 2.9305x over previous
import functools
import math

import jax
import jax.numpy as jnp
from jax import lax
from jax.experimental import pallas as pl
from jax.experimental.pallas import tpu as pltpu
from jax.experimental.pallas import tpu_sc as plsc

_NC = 2
_NS = 16
_NW = _NC * _NS


def _chunk(n, cap, align=8):
    for c in range(min(cap, n), 0, -1):
        if c % align == 0 and n % c == 0:
            return c
    raise ValueError(f"no chunk for n={n} cap={cap}")


def _out_split(rows):
    for t in range(_NS, 0, -1):
        if rows % t == 0 and (rows // t) % 8 == 0:
            return t, rows // t
    raise ValueError(f"no 8-aligned split for {rows}")


def _mesh():
    return plsc.VectorSubcoreMesh(core_axis_name="c", subcore_axis_name="s")


_SC_PARAMS = pltpu.CompilerParams(use_tc_tiling_on_sc=False)


def _sc_gather(table, idx, chunk_cap=80):
    R, D = table.shape
    (M,) = idx.shape
    per_w = M // _NW
    assert per_w * _NW == M and per_w % 8 == 0, (M, per_w)
    CH = _chunk(per_w, chunk_cap)
    n_iter = per_w // CH

    @functools.partial(
        pl.kernel,
        out_type=jax.ShapeDtypeStruct((M, D), table.dtype),
        mesh=_mesh(),
        compiler_params=_SC_PARAMS,
        scratch_types=[
            pltpu.VMEM((CH,), jnp.int32),
            pltpu.VMEM((CH, D), table.dtype),
            pltpu.SemaphoreType.DMA,
        ],
    )
    def gk(table_hbm, idx_hbm, out_hbm, idx_v, rows_v, sem):
        wid = lax.axis_index("s") * _NC + lax.axis_index("c")
        base = wid * per_w

        def body(i, carry):
            off = base + i * CH
            pltpu.sync_copy(idx_hbm.at[pl.ds(off, CH)], idx_v)
            pltpu.async_copy(table_hbm.at[idx_v], rows_v, sem).wait()
            pltpu.sync_copy(rows_v, out_hbm.at[pl.ds(off, CH)])
            return carry

        lax.fori_loop(0, n_iter, body, 0)

    return gk(table, idx)


def _sc_zero_rows(buf, n_rows, width):
    zero = jnp.zeros((16,), jnp.float32)

    def z(i, carry):
        for c in range(width // 16):
            buf[i, pl.ds(c * 16, 16)] = zero
        return carry

    lax.fori_loop(0, n_rows, z, 0)


def _sc_scatter16(rows, idx, row_off, Mt, R, chunk_cap=80):
    per_t = Mt // _NS
    assert per_t * _NS == Mt and per_t % 8 == 0
    CH = _chunk(per_t, chunk_cap)
    assert CH % 16 == 0
    n_iter = per_t // CH
    half = R // 2
    assert half * 2 == R and half % 8 == 0
    n_out, rows_o = _out_split(half)
    ZCH = _chunk(rows_o, 256)

    @functools.partial(
        pl.kernel,
        out_type=jax.ShapeDtypeStruct((R, 16), jnp.float32),
        mesh=_mesh(),
        compiler_params=_SC_PARAMS,
        scratch_types=[
            pltpu.VMEM((CH,), jnp.int32),
            pltpu.VMEM((CH, 16), jnp.float32),
            pltpu.VMEM((ZCH, 16), jnp.float32),
            pltpu.VMEM_SHARED((half + 8, 16), jnp.float32),
            pltpu.SemaphoreType.DMA,
        ],
    )
    def sk(rows_hbm, idx_hbm, out_hbm, idx_v, rows_v, zb, acc_sp, sem):
        cid = lax.axis_index("c")
        sid = lax.axis_index("s")

        @pl.when(sid < n_out)
        def _():
            _sc_zero_rows(zb, ZCH, 16)
            for j in range(rows_o // ZCH):
                pltpu.sync_copy(
                    zb, acc_sp.at[pl.ds(sid * rows_o + j * ZCH, ZCH)])

        @pl.when(sid == (n_out % _NS))
        def _():
            _sc_zero_rows(zb, 8, 16)
            pltpu.sync_copy(zb.at[pl.ds(0, 8)], acc_sp.at[pl.ds(half, 8)])

        plsc.subcore_barrier()

        lo = cid * half
        base = row_off + sid * per_t

        def body(i, carry):
            off = base + i * CH
            pltpu.sync_copy(idx_hbm.at[pl.ds(off, CH)], idx_v)
            pltpu.sync_copy(rows_hbm.at[pl.ds(off, CH)], rows_v)

            def clamp(j, c2):
                v = idx_v[pl.ds(j * 16, 16)]
                local = v - lo
                valid = (local >= 0) & (local < half)
                idx_v[pl.ds(j * 16, 16)] = jnp.where(valid, local, half)
                return c2

            lax.fori_loop(0, CH // 16, clamp, 0)
            pltpu.sync_copy(rows_v, acc_sp.at[idx_v], add=True)
            return carry

        lax.fori_loop(0, n_iter, body, 0)
        plsc.subcore_barrier()

        @pl.when(sid < n_out)
        def _():
            pltpu.sync_copy(
                acc_sp.at[pl.ds(sid * rows_o, rows_o)],
                out_hbm.at[pl.ds(cid * half + sid * rows_o, rows_o)],
            )

    return sk(rows, idx)


def _sc_scatter_msgs(msg_c, msg_s, dst, row_off, Et, N, chunk_cap=80):
    per_t = Et // _NS
    assert per_t * _NS == Et and per_t % 8 == 0
    CH = _chunk(per_t, chunk_cap)
    n_iter = per_t // CH
    n_out, rows_o = _out_split(N)
    ZCH = _chunk(rows_o, 256)

    @functools.partial(
        pl.kernel,
        out_type=(
            jax.ShapeDtypeStruct((N, 128), jnp.float32),
            jax.ShapeDtypeStruct((N, 128), jnp.float32),
        ),
        mesh=_mesh(),
        compiler_params=_SC_PARAMS,
        scratch_types=[
            pltpu.VMEM((CH,), jnp.int32),
            pltpu.VMEM((CH, 128), jnp.float32),
            pltpu.VMEM((ZCH, 128), jnp.float32),
            pltpu.VMEM_SHARED((N, 128), jnp.float32),
            pltpu.SemaphoreType.DMA,
        ],
    )
    def sk(mc_hbm, ms_hbm, dst_hbm, u_hbm, s_hbm,
           idx_v, rows_v, zb, acc_sp, sem):
        cid = lax.axis_index("c")
        sid = lax.axis_index("s")

        @pl.when(sid < n_out)
        def _():
            _sc_zero_rows(zb, ZCH, 128)
            for j in range(rows_o // ZCH):
                pltpu.sync_copy(
                    zb, acc_sp.at[pl.ds(sid * rows_o + j * ZCH, ZCH)])

        plsc.subcore_barrier()

        base = row_off + sid * per_t

        def body(i, carry):
            off = base + i * CH
            pltpu.sync_copy(dst_hbm.at[pl.ds(off, CH)], idx_v)

            @pl.when(cid == 0)
            def _():
                pltpu.sync_copy(mc_hbm.at[pl.ds(off, CH)], rows_v)

            @pl.when(cid == 1)
            def _():
                pltpu.sync_copy(ms_hbm.at[pl.ds(off, CH)], rows_v)

            pltpu.sync_copy(rows_v, acc_sp.at[idx_v], add=True)
            return carry

        lax.fori_loop(0, n_iter, body, 0)
        plsc.subcore_barrier()
        sl = pl.ds(sid * rows_o, rows_o)

        @pl.when((cid == 0) & (sid < n_out))
        def _():
            pltpu.sync_copy(acc_sp.at[sl], u_hbm.at[sl])

        @pl.when((cid == 1) & (sid < n_out))
        def _():
            pltpu.sync_copy(acc_sp.at[sl], s_hbm.at[sl])

    return sk(msg_c, msg_s, dst)


def _tc_qkv(x_all, w_cat, b_cat):
    TN, D = x_all.shape
    BL = _chunk(TN, 3000)
    grid = (TN // BL,)

    def body(x_ref, w_ref, b_ref, q_ref, k_ref, v_ref):
        y = jnp.dot(x_ref[...], w_ref[...],
                    preferred_element_type=jnp.float32) + b_ref[...]
        q_ref[...] = y[:, 0:128]
        k_ref[...] = y[:, 128:256]
        v_ref[...] = y[:, 256:384]

    out = jax.ShapeDtypeStruct((TN, 128), jnp.float32)
    return pl.pallas_call(
        body,
        grid=grid,
        in_specs=[
            pl.BlockSpec((BL, D), lambda i: (i, 0)),
            pl.BlockSpec((D, 384), lambda i: (0, 0)),
            pl.BlockSpec((1, 384), lambda i: (0, 0)),
        ],
        out_specs=[pl.BlockSpec((BL, 128), lambda i: (i, 0))] * 3,
        out_shape=[out, out, out],
    )(x_all, w_cat, b_cat)


def _src_of_pair(p, T):
    tt = 0
    for k in range(1, T):
        tt = tt + (p >= k * (k + 1) // 2).astype(jnp.int32)
    return p - tt * (tt + 1) // 2


def _tc_z(qg, kg, r_sum, T, E, BL):
    M = qg.shape[0]
    n_pair = M // E
    nb = E // BL

    def body(q_ref, k_ref, r_ref, z_ref):
        prod = q_ref[...] * k_ref[...]
        att = jnp.dot(prod, r_ref[...], preferred_element_type=jnp.float32)
        lane = lax.broadcasted_iota(jnp.int32, att.shape, 1)
        z_ref[...] = jnp.where(lane < 8, jnp.exp(att), 0.0)

    return pl.pallas_call(
        body,
        grid=(n_pair, nb),
        in_specs=[
            pl.BlockSpec((BL, 128), lambda p, c: (p * nb + c, 0)),
            pl.BlockSpec((BL, 128),
                         lambda p, c: (_src_of_pair(p, T) * nb + c, 0)),
            pl.BlockSpec((128, 16), lambda p, c: (0, 0)),
        ],
        out_specs=pl.BlockSpec((BL, 16), lambda p, c: (p * nb + c, 0)),
        out_shape=jax.ShapeDtypeStruct((M, 16), jnp.float32),
    )(qg, kg, r_sum)


def _tc_phase_b(z, deng, vg, r16, T, E, BL):
    M = z.shape[0]
    n_pair = M // E
    nb = E // BL

    def body(z_ref, d_ref, v_ref, r_ref, e2_ref, mc_ref, ms_ref):
        zb = z_ref[...]
        res = zb / (d_ref[...] + 1e-16)
        lane = lax.broadcasted_iota(jnp.int32, res.shape, 1)
        e2 = jnp.where(lane < 8, jnp.exp(-res), 0.0)
        e2_ref[...] = e2
        vgb = v_ref[...]
        r = r_ref[...]
        mc_ref[...] = jnp.dot(zb, r, preferred_element_type=jnp.float32) * vgb
        ms_ref[...] = jnp.dot(e2, r, preferred_element_type=jnp.float32) * vgb

    o16 = jax.ShapeDtypeStruct((M, 16), jnp.float32)
    o128 = jax.ShapeDtypeStruct((M, 128), jnp.float32)
    return pl.pallas_call(
        body,
        grid=(n_pair, nb),
        in_specs=[
            pl.BlockSpec((BL, 16), lambda p, c: (p * nb + c, 0)),
            pl.BlockSpec((BL, 16), lambda p, c: (p * nb + c, 0)),
            pl.BlockSpec((BL, 128),
                         lambda p, c: (_src_of_pair(p, T) * nb + c, 0)),
            pl.BlockSpec((16, 128), lambda p, c: (0, 0)),
        ],
        out_specs=[
            pl.BlockSpec((BL, 16), lambda p, c: (p * nb + c, 0)),
            pl.BlockSpec((BL, 128), lambda p, c: (p * nb + c, 0)),
            pl.BlockSpec((BL, 128), lambda p, c: (p * nb + c, 0)),
        ],
        out_shape=[o16, o128, o128],
    )(z, deng, vg, r16)


def _erf(x):
    t = 1.0 / (1.0 + 0.3275911 * jnp.abs(x))
    poly = ((((1.061405429 * t - 1.453152027) * t + 1.421413741) * t
             - 0.284496736) * t + 0.254829592) * t
    return jnp.sign(x) * (1.0 - poly * jnp.exp(-x * x))


def _tc_final(u, s, den, den2, x_tar, r16, w1, b1, w2, b2, g, beta):
    N = u.shape[0]
    BL = _chunk(N, 2000)
    inv_s2 = 1.0 / math.sqrt(2.0)

    def body(u_ref, s_ref, dn_ref, d2_ref, x_ref, r_ref, w1_ref, b1_ref,
             w2_ref, b2_ref, g_ref, be_ref, xs_ref, cs_ref, ss_ref):
        r = r_ref[...]
        den_rep = jnp.dot(dn_ref[...], r, preferred_element_type=jnp.float32)
        d2_rep = jnp.dot(d2_ref[...], r, preferred_element_type=jnp.float32)
        ch = u_ref[...] / (den_rep + 1e-16) + x_ref[...]
        sh = s_ref[...] / (d2_rep + 1e-16)

        def ffn(x):
            mu = jnp.mean(x, axis=-1, keepdims=True)
            var = jnp.mean((x - mu) ** 2, axis=-1, keepdims=True)
            h = (x - mu) / jnp.sqrt(var + 1e-5) * g_ref[...] + be_ref[...]
            h = jnp.dot(h, w1_ref[...], preferred_element_type=jnp.float32) \
                + b1_ref[...]
            h = 0.5 * h * (1.0 + _erf(h * inv_s2))
            h = jnp.dot(h, w2_ref[...], preferred_element_type=jnp.float32) \
                + b2_ref[...]
            return x + h

        c = ffn(ch)
        sp = ffn(sh)
        xs_ref[...] = c + sp
        cs_ref[...] = c
        ss_ref[...] = sp

    o = jax.ShapeDtypeStruct((N, 128), jnp.float32)
    blk = lambda shp: pl.BlockSpec(shp, lambda i: (0, 0))
    return pl.pallas_call(
        body,
        grid=(N // BL,),
        in_specs=[
            pl.BlockSpec((BL, 128), lambda i: (i, 0)),
            pl.BlockSpec((BL, 128), lambda i: (i, 0)),
            pl.BlockSpec((BL, 16), lambda i: (i, 0)),
            pl.BlockSpec((BL, 16), lambda i: (i, 0)),
            pl.BlockSpec((BL, 128), lambda i: (i, 0)),
            blk((16, 128)), blk((128, 256)), blk((1, 256)),
            blk((256, 128)), blk((1, 128)), blk((1, 128)), blk((1, 128)),
        ],
        out_specs=[pl.BlockSpec((BL, 128), lambda i: (i, 0))] * 3,
        out_shape=[o, o, o],
    )(u, s, den, den2, x_tar, r16, w1, b1, w2, b2, g, beta)


def kernel(x_list, edge_index_list, Wq, bq, Wk, bk, Wv, bv, ln_g, ln_b,
           W1, b1, W2, b2):
    T, N, D = x_list.shape
    E = edge_index_list.shape[2]
    H, DK = 8, 16
    pairs = [(tt, s) for tt in range(T) for s in range(tt + 1)]
    BL = _chunk(E, 2560)

    r_np = jnp.zeros((128, 16), jnp.float32)
    hh = jnp.arange(128) // DK
    r_sum = r_np.at[jnp.arange(128), hh].set(1.0 / math.sqrt(DK))
    r16 = jnp.zeros((16, 128), jnp.float32).at[hh, jnp.arange(128)].set(1.0)

    x_all = x_list.reshape(T * N, D)
    w_cat = jnp.concatenate([Wq, Wk, Wv], axis=1)
    b_cat = jnp.concatenate([bq, bk, bv]).reshape(1, 384)
    QT, KT, VT = _tc_qkv(x_all, w_cat, b_cat)

    src = edge_index_list[:, 0, :]
    dst = edge_index_list[:, 1, :]
    qidx = jnp.concatenate([tt * N + dst[s] for (tt, s) in pairs])
    kidx = (jnp.arange(T, dtype=jnp.int32)[:, None] * N + src).reshape(T * E)
    dst_all = jnp.concatenate([dst[s] for (_, s) in pairs])

    qg = _sc_gather(QT, qidx)
    kg = _sc_gather(KT, kidx)
    vg = _sc_gather(VT, kidx)
    z = _tc_z(qg, kg, r_sum, T, E, BL)
    den = _sc_scatter16(z, qidx, 0, z.shape[0], T * N)

    deng = _sc_gather(den, qidx)
    e2, msg_c, msg_s = _tc_phase_b(z, deng, vg, r16, T, E, BL)

    xs, cs, ss = [], [], []
    row_off = 0
    for tt in range(T):
        Et = (tt + 1) * E
        u, s_hat = _sc_scatter_msgs(msg_c, msg_s, dst_all, row_off, Et, N)
        den2 = _sc_scatter16(e2, dst_all, row_off, Et, N)
        xo, co, so = _tc_final(
            u, s_hat, lax.dynamic_slice_in_dim(den, tt * N, N), den2,
            x_list[tt], r16, W1, b1.reshape(1, 256), W2, b2.reshape(1, 128),
            ln_g.reshape(1, 128), ln_b.reshape(1, 128))
        xs.append(xo)
        cs.append(co)
        ss.append(so)
        row_off += Et

    return jnp.stack(xs), jnp.stack(cs), jnp.stack(ss)
